# baseline (speedup 1.0000x reference)
import functools

import jax
import jax.numpy as jnp
from jax import lax
from jax.experimental import pallas as pl
from jax.experimental.pallas import tpu as pltpu
from jax.experimental.pallas import tpu_sc as plsc

VOCAB = 100000
EMBED = 128
SEQ = 200
HID = 256
BATCH = 1024
B2 = 2 * BATCH

_NC, _NS = 2, 16
NW = _NC * _NS
NCHUNK = 5
TCH = SEQ // NCHUNK
CH_ROWS = B2 * TCH
CH_IDX_ROWS = CH_ROWS // 128
ROWS_PER_W = CH_IDX_ROWS // NW
CHUNK = 128


def _sc_gather(table, idx_flat):
    idx3d = idx_flat.reshape(NW, ROWS_PER_W, 128)
    mesh = plsc.VectorSubcoreMesh(core_axis_name="c", subcore_axis_name="s")

    @functools.partial(
        pl.kernel,
        mesh=mesh,
        out_type=jax.ShapeDtypeStruct((CH_ROWS, EMBED), jnp.float32),
        scratch_types=[
            pltpu.VMEM((ROWS_PER_W, CHUNK), jnp.int32),
            pltpu.VMEM((2 * CHUNK, EMBED), jnp.float32),
            pltpu.VMEM((2 * CHUNK, EMBED), jnp.float32),
            pltpu.SemaphoreType.DMA,
            pltpu.SemaphoreType.DMA,
        ],
    )
    def k(table_hbm, idx_hbm, out_hbm, idx_v, rows_a, rows_b, sem_a, sem_b):
        wid = lax.axis_index("s") * _NC + lax.axis_index("c")
        base = wid * ROWS_PER_W
        pltpu.sync_copy(idx_hbm.at[wid], idx_v)

        def fill(buf, sem, j):
            pltpu.async_copy(table_hbm.at[idx_v.at[j]],
                             buf.at[pl.ds(0, CHUNK)], sem)
            pltpu.async_copy(table_hbm.at[idx_v.at[j + 1]],
                             buf.at[pl.ds(CHUNK, CHUNK)], sem)

        def drain_store(buf, sem, j):
            pltpu.make_async_copy(table_hbm.at[pl.ds(0, 2 * CHUNK)], buf,
                                  sem).wait()
            pltpu.sync_copy(buf,
                            out_hbm.at[pl.ds((base + j) * CHUNK, 2 * CHUNK)])

        fill(rows_a, sem_a, 0)
        fill(rows_b, sem_b, 2)

        def body(jj, carry):
            j = 4 * jj
            drain_store(rows_a, sem_a, j)

            @pl.when(j + 4 < ROWS_PER_W)
            def _pa():
                fill(rows_a, sem_a, j + 4)

            drain_store(rows_b, sem_b, j + 2)

            @pl.when(j + 6 < ROWS_PER_W)
            def _pb():
                fill(rows_b, sem_b, j + 6)

            return carry

        lax.fori_loop(0, ROWS_PER_W // 4, body, 0)

    return k(table, idx3d)


def _make_lstm_chunk(last):
    RT = min(64, B2)
    SL = min(512, B2)
    NSL = B2 // SL
    NCT = HID // 128

    def body(x_ref, idx_ref, wu_ref, b_ref, h0_ref, c0_ref, *refs):
        if last:
            c1_ref, c2_ref, sim_ref, c_s = refs[:4]
            rest = refs[4:]
        else:
            ho_ref, co_ref, c_s = refs[:3]
            rest = refs[3:]
        xh_bufs = rest[0:NSL]
        h_bufs = rest[NSL:2 * NSL]
        zbufs = rest[2 * NSL:3 * NSL]
        t = pl.program_id(0)

        @pl.when(t == 0)
        def _init():
            c_s[...] = c0_ref[...]
            for s in range(NSL):
                h_bufs[s][...] = h0_ref[pl.ds(s * SL, SL), :]

        def tile(zbuf, hbuf, s, bt):
            rs = pl.ds(s * SL + bt * RT, RT)
            zrs = pl.ds(bt * RT, RT)
            m = idx_ref[0, rs, :] != 0
            for kc in range(NCT):
                col = pl.ds(kc * 128, 128)
                zi = zbuf[zrs, pl.ds(0 * HID + kc * 128, 128)] + b_ref[0, pl.ds(0 * HID + kc * 128, 128)]
                zf = zbuf[zrs, pl.ds(1 * HID + kc * 128, 128)] + b_ref[0, pl.ds(1 * HID + kc * 128, 128)]
                zg = zbuf[zrs, pl.ds(2 * HID + kc * 128, 128)] + b_ref[0, pl.ds(2 * HID + kc * 128, 128)]
                zo = zbuf[zrs, pl.ds(3 * HID + kc * 128, 128)] + b_ref[0, pl.ds(3 * HID + kc * 128, 128)]
                i = 0.5 * jnp.tanh(zi) + 0.5
                f = 0.5 * jnp.tanh(zf) + 0.5
                g = jnp.tanh(zg)
                o = 0.5 * jnp.tanh(zo) + 0.5
                c_old = c_s[rs, col]
                c_new = f * c_old + i * g
                h_new = o * jnp.tanh(c_new)
                c_s[rs, col] = jnp.where(m, c_new, c_old)
                h_bufs[s][zrs, col] = jnp.where(m, h_new.astype(jnp.bfloat16),
                                                h_bufs[s][zrs, col])

        for s in range(NSL):
            xh = xh_bufs[s]
            xh[:, :EMBED] = x_ref[0, pl.ds(s * SL, SL), :].astype(jnp.bfloat16)
            xh[:, EMBED:] = h_bufs[s][...]
        for s in range(NSL):
            zbufs[s][...] = jnp.dot(xh_bufs[s][...], wu_ref[...],
                                    preferred_element_type=jnp.float32)
            for bt in range(SL // RT):
                tile(zbufs[s], h_bufs[s], s, bt)

        @pl.when(t == TCH - 1)
        def _fin():
            if last:
                cc = c_s[...]
                a = cc[:BATCH]
                bb = cc[BATCH:]
                na = jnp.sum(a * a, axis=1, keepdims=True)
                nb = jnp.sum(bb * bb, axis=1, keepdims=True)
                ab = jnp.sum(a * bb, axis=1, keepdims=True)
                inv = lax.rsqrt(jnp.maximum(na, 1e-12) * jnp.maximum(nb, 1e-12))
                c1_ref[...] = a
                c2_ref[...] = bb
                sim_ref[...] = ab * inv
            else:
                for s in range(NSL):
                    ho_ref[pl.ds(s * SL, SL), :] = h_bufs[s][...]
                co_ref[...] = c_s[...]

    if last:
        out_specs = [
            pl.BlockSpec((BATCH, HID), lambda t: (0, 0)),
            pl.BlockSpec((BATCH, HID), lambda t: (0, 0)),
            pl.BlockSpec((BATCH, 1), lambda t: (0, 0)),
        ]
        out_shape = [
            jax.ShapeDtypeStruct((BATCH, HID), jnp.float32),
            jax.ShapeDtypeStruct((BATCH, HID), jnp.float32),
            jax.ShapeDtypeStruct((BATCH, 1), jnp.float32),
        ]
    else:
        out_specs = [
            pl.BlockSpec((B2, HID), lambda t: (0, 0)),
            pl.BlockSpec((B2, HID), lambda t: (0, 0)),
        ]
        out_shape = [
            jax.ShapeDtypeStruct((B2, HID), jnp.bfloat16),
            jax.ShapeDtypeStruct((B2, HID), jnp.float32),
        ]

    return pl.pallas_call(
        body,
        grid=(TCH,),
        in_specs=[
            pl.BlockSpec((1, B2, EMBED), lambda t: (t, 0, 0)),
            pl.BlockSpec((1, B2, 1), lambda t: (t, 0, 0)),
            pl.BlockSpec((EMBED + HID, 4 * HID), lambda t: (0, 0)),
            pl.BlockSpec((1, 4 * HID), lambda t: (0, 0)),
            pl.BlockSpec((B2, HID), lambda t: (0, 0)),
            pl.BlockSpec((B2, HID), lambda t: (0, 0)),
        ],
        out_specs=out_specs,
        out_shape=out_shape,
        scratch_shapes=(
            [pltpu.VMEM((B2, HID), jnp.float32)]
            + [pltpu.VMEM((SL, EMBED + HID), jnp.bfloat16)
               for _ in range(NSL)]
            + [pltpu.VMEM((SL, HID), jnp.bfloat16) for _ in range(NSL)]
            + [pltpu.VMEM((SL, 4 * HID), jnp.float32) for _ in range(NSL)]
        ),
    )


def kernel(funcname_1, funcname_2, table, W, U, b):
    idx = jnp.concatenate([funcname_1, funcname_2], axis=0).astype(jnp.int32)
    idxT = idx.T

    scale = jnp.concatenate([
        jnp.full((HID,), 0.5, jnp.float32),
        jnp.full((HID,), 0.5, jnp.float32),
        jnp.ones((HID,), jnp.float32),
        jnp.full((HID,), 0.5, jnp.float32),
    ])
    wu = (jnp.concatenate([W, U], axis=0) * scale[None, :]).astype(jnp.bfloat16)
    bs = (b * scale).reshape(1, 4 * HID)

    h = jnp.zeros((B2, HID), jnp.bfloat16)
    c = jnp.zeros((B2, HID), jnp.float32)
    xs = [
        _sc_gather(table, idxT[k * TCH:(k + 1) * TCH].reshape(-1))
        for k in range(NCHUNK)
    ]
    for k in range(NCHUNK):
        x_k = xs[k].reshape(TCH, B2, EMBED)
        idx_k = idxT[k * TCH:(k + 1) * TCH].reshape(TCH, B2, 1)
        if k < NCHUNK - 1:
            h, c = _make_lstm_chunk(False)(x_k, idx_k, wu, bs, h, c)
        else:
            c1, c2, sim = _make_lstm_chunk(True)(x_k, idx_k, wu, bs, h, c)
    return (c1, c2, sim.reshape(BATCH))
